# baseline (speedup 1.0000x reference)
import functools

import jax
import jax.numpy as jnp
from jax import lax
from jax.experimental import pallas as pl
from jax.experimental.pallas import tpu as pltpu
from jax.experimental.pallas import tpu_sc as plsc

N = 10000
R = 4000
G = 64
EL = 320000
EC = 160000
ECP = 163840
D = 128
NR = 32
CUT = 5.0
GAMMA = 10.0

NC = 2
NS = 16
BL = 80

Q = 32768
DMAX = 8.0
SCALE = Q / DMAX
HI = float(Q - 1) - 1e-3
QV = 8192
SCALEV = QV / DMAX
HIV = float(QV - 1) - 1e-3

_HIGH = jax.lax.Precision.HIGHEST


def _dot(a, b):
    return jnp.dot(a, b, precision=_HIGH, preferred_element_type=jnp.float32)


def _embed_body(z_ref, emb_ref, o_ref, *, k):
    z = z_ref[...]
    ks = lax.broadcasted_iota(jnp.int32, (z.shape[0], k), 1)
    onehot = (ks == z).astype(jnp.float32)
    o_ref[...] = _dot(onehot, emb_ref[...])


def _tc_embed(z, emb, n, k, blk):
    grid = n // blk
    return pl.pallas_call(
        functools.partial(_embed_body, k=k),
        grid=(grid,),
        in_specs=[
            pl.BlockSpec((blk, 1), lambda i: (i, 0)),
            pl.BlockSpec((k, D), lambda i: (0, 0)),
        ],
        out_specs=pl.BlockSpec((blk, D), lambda i: (i, 0)),
        out_shape=jax.ShapeDtypeStruct((n, D), jnp.float32),
    )(z.reshape(n, 1), emb)


def _h_body(rt_ref, emb_ref, pn_ref, pca_ref, pc_ref, wg_ref, bg_ref, o_ref):
    rt = rt_ref[...]
    ks = lax.broadcasted_iota(jnp.int32, (R, 26), 1)
    he = _dot((ks == rt).astype(jnp.float32), emb_ref[...])
    v1 = pn_ref[...] - pca_ref[...]
    v2 = pc_ref[...] - pca_ref[...]
    n1 = jnp.sqrt(jnp.sum(v1 * v1, axis=1, keepdims=True) + 1e-6)
    n2 = jnp.sqrt(jnp.sum(v2 * v2, axis=1, keepdims=True) + 1e-6)
    d12 = jnp.sum(v1 * v2, axis=1, keepdims=True)
    geom = jnp.concatenate([n1, n2, d12], axis=1)
    o_ref[...] = he + jnp.tanh(_dot(geom, wg_ref[...]) + bg_ref[...])


def _tc_h(rt, emb_res, pn, pca, pc, wg, bg):
    return pl.pallas_call(
        _h_body,
        out_shape=jax.ShapeDtypeStruct((R, D), jnp.float32),
    )(rt.reshape(R, 1), emb_res, pn, pca, pc, wg, bg.reshape(1, D))


def _rbf_rows(rows, centers, scale):
    d = rows * jnp.float32(1.0 / scale)
    return jnp.exp(-GAMMA * (d[:, None] - centers) ** 2)


def _tab1_body(cen_ref, w0_ref, w1_ref, t0_ref, t1_ref, *, blk):
    pid = pl.program_id(0)
    rows = (lax.broadcasted_iota(jnp.int32, (blk,), 0)
            + pid * blk).astype(jnp.float32)
    r = _rbf_rows(rows, cen_ref[...], SCALE)
    t0_ref[...] = _dot(r, w0_ref[...])
    t1_ref[...] = _dot(r, w1_ref[...])


def _tc_tab1(centers, w0, w1, blk=4096):
    return pl.pallas_call(
        functools.partial(_tab1_body, blk=blk),
        grid=(Q // blk,),
        in_specs=[
            pl.BlockSpec((1, NR), lambda i: (0, 0)),
            pl.BlockSpec((NR, D), lambda i: (0, 0)),
            pl.BlockSpec((NR, D), lambda i: (0, 0)),
        ],
        out_specs=[
            pl.BlockSpec((blk, D), lambda i: (i, 0)),
            pl.BlockSpec((blk, D), lambda i: (i, 0)),
        ],
        out_shape=[
            jax.ShapeDtypeStruct((Q, D), jnp.float32),
            jax.ShapeDtypeStruct((Q, D), jnp.float32),
        ],
    )(centers, w0, w1)


def _tab2_body(cen_ref, w0_ref, w1_ref, tc_ref, *, blk, half):
    pid = pl.program_id(0)
    rows = (lax.broadcasted_iota(jnp.int32, (blk,), 0)
            + (pid % half) * blk).astype(jnp.float32)
    r = _rbf_rows(rows, cen_ref[...], SCALE)
    w = jnp.where(pid < half, w0_ref[...], w1_ref[...])
    tc_ref[...] = _dot(r, w)


def _tc_tab2(centers, w0, w1, blk=4096):
    half = Q // blk
    return pl.pallas_call(
        functools.partial(_tab2_body, blk=blk, half=half),
        grid=(2 * half,),
        in_specs=[
            pl.BlockSpec((1, NR), lambda i: (0, 0)),
            pl.BlockSpec((NR, D), lambda i: (0, 0)),
            pl.BlockSpec((NR, D), lambda i: (0, 0)),
        ],
        out_specs=pl.BlockSpec((blk, D), lambda i: (i, 0)),
        out_shape=jax.ShapeDtypeStruct((2 * Q, D), jnp.float32),
    )(centers, w0, w1)


def _vtab_body(cen_ref, wv_ref, vt_ref, *, blk):
    pid = pl.program_id(0)
    rows = (lax.broadcasted_iota(jnp.int32, (blk,), 0)
            + pid * blk).astype(jnp.float32)
    r = _rbf_rows(rows, cen_ref[...], SCALEV)
    vt_ref[...] = _dot(r, wv_ref[...])


def _tc_vtab(centers, wv, blk=4096):
    return pl.pallas_call(
        functools.partial(_vtab_body, blk=blk),
        grid=(QV // blk,),
        in_specs=[
            pl.BlockSpec((1, NR), lambda i: (0, 0)),
            pl.BlockSpec((NR, 1), lambda i: (0, 0)),
        ],
        out_specs=pl.BlockSpec((blk, 1), lambda i: (i, 0)),
        out_shape=jax.ShapeDtypeStruct((QV, 1), jnp.float32),
    )(centers, wv)


def _upd_body(x_ref, a0_ref, a1_ref, w_ref, b_ref, o_ref):
    agg = a0_ref[...] + a1_ref[...]
    o_ref[...] = x_ref[...] + jnp.tanh(_dot(agg, w_ref[...]) + b_ref[...])


def _tc_upd(x, a0, a1, w, b, blk=1000):
    return pl.pallas_call(
        _upd_body,
        grid=(N // blk,),
        in_specs=[
            pl.BlockSpec((blk, D), lambda i: (i, 0)),
            pl.BlockSpec((blk, D), lambda i: (i, 0)),
            pl.BlockSpec((blk, D), lambda i: (i, 0)),
            pl.BlockSpec((D, D), lambda i: (0, 0)),
            pl.BlockSpec((1, D), lambda i: (0, 0)),
        ],
        out_specs=pl.BlockSpec((blk, D), lambda i: (i, 0)),
        out_shape=jax.ShapeDtypeStruct((N, D), jnp.float32),
    )(x, a0, a1, w, b.reshape(1, D))


def _final_body(x_ref, c0_ref, c1_ref, va_ref, vb_ref, w0_ref, b0_ref,
                w1_ref, b1_ref, o_ref):
    v = va_ref[...] + vb_ref[...]
    wx = v[:, 0:1]
    wy = v[:, 1:2]
    wz = v[:, 2:3]
    vn = jnp.sqrt(wx * wx + wy * wy + wz * wz + 1e-6)
    m = 1.0 + vn
    x = x_ref[...] + jnp.tanh(_dot(c0_ref[...] * m, w0_ref[...]) + b0_ref[...])
    o_ref[...] = x + jnp.tanh(_dot(c1_ref[...] * m, w1_ref[...]) + b1_ref[...])


def _tc_final(x, c0, c1, va, vb, w0, b0, w1, b1, blk=1000):
    bspec = pl.BlockSpec((blk, D), lambda i: (i, 0))
    return pl.pallas_call(
        _final_body,
        grid=(N // blk,),
        in_specs=[
            bspec, bspec, bspec, bspec, bspec,
            pl.BlockSpec((D, D), lambda i: (0, 0)),
            pl.BlockSpec((1, D), lambda i: (0, 0)),
            pl.BlockSpec((D, D), lambda i: (0, 0)),
            pl.BlockSpec((1, D), lambda i: (0, 0)),
        ],
        out_specs=bspec,
        out_shape=jax.ShapeDtypeStruct((N, D), jnp.float32),
    )(x, c0, c1, va, vb, w0, b0.reshape(1, D), w1, b1.reshape(1, D))


_MESH = dict(core_axis_name="c", subcore_axis_name="s")
_NOLAYOUT = pltpu.CompilerParams(needs_layout_passes=False)


def _rsqrt16(s):
    bits = plsc.bitcast(s, jnp.int32)
    y = plsc.bitcast(jnp.int32(0x5F3759DF) - lax.shift_right_logical(bits, 1),
                     jnp.float32)
    for _ in range(3):
        y = y * (1.5 - 0.5 * s * y * y)
    return y


def _sc_geo():
    pwl = EL // (NC * NS)
    nbl = pwl // BL
    pwc = ECP // (NC * NS)
    nbc = pwc // BL

    def body(pxl, pyl, pzl, pcx, pcy, pcz, vt_hbm, srcl, dstl, lip, rip,
             ql, tl, vmx, vmy, vmz, qc, tc,
             pxv, pyv, pzv, cxv, cyv, czv, vtv,
             a_v, b_v, q_v, t_v, mx_v, my_v, mz_v, sem):
        c = lax.axis_index("c")
        s = lax.axis_index("s")
        pltpu.sync_copy(pxl, pxv)
        pltpu.sync_copy(pyl, pyv)
        pltpu.sync_copy(pzl, pzv)
        pltpu.sync_copy(pcx, cxv)
        pltpu.sync_copy(pcy, cyv)
        pltpu.sync_copy(pcz, czv)
        pltpu.sync_copy(vt_hbm, vtv)
        wid = c * NS + s

        def lblock(b, _):
            base = wid * pwl + b * BL
            pltpu.sync_copy(srcl.at[pl.ds(base, BL)], a_v)
            pltpu.sync_copy(dstl.at[pl.ds(base, BL)], b_v)
            for g in range(BL // 16):
                sl = pl.ds(g * 16, 16)
                sv = a_v[sl]
                dv = b_v[sl]
                dx = plsc.load_gather(pxv, [sv]) - plsc.load_gather(pxv, [dv])
                dy = plsc.load_gather(pyv, [sv]) - plsc.load_gather(pyv, [dv])
                dz = plsc.load_gather(pzv, [sv]) - plsc.load_gather(pzv, [dv])
                ss = dx * dx + dy * dy + dz * dz + 1e-6
                yv = _rsqrt16(ss)
                d = ss * yv
                f = jnp.minimum(d * jnp.float32(SCALE), HI)
                q = f.astype(jnp.int32)
                q_v[sl] = q
                t_v[sl] = f - q.astype(jnp.float32)
                fv = jnp.minimum(d * jnp.float32(SCALEV), HIV)
                qv = fv.astype(jnp.int32)
                tv = fv - qv.astype(jnp.float32)
                v0 = plsc.load_gather(vtv, [qv])
                v1 = plsc.load_gather(vtv, [qv + 1])
                vv = (v0 + (v1 - v0) * tv) * yv
                mx_v[sl] = dx * vv
                my_v[sl] = dy * vv
                mz_v[sl] = dz * vv
            osl = pl.ds(base, BL)
            pltpu.sync_copy(q_v, ql.at[osl])
            pltpu.sync_copy(t_v, tl.at[osl])
            pltpu.sync_copy(mx_v, vmx.at[osl])
            pltpu.sync_copy(my_v, vmy.at[osl])
            pltpu.sync_copy(mz_v, vmz.at[osl])
            return 0

        lax.fori_loop(0, nbl, lblock, 0)

        def cblock(b, _):
            base = wid * pwc + b * BL
            pltpu.sync_copy(lip.at[pl.ds(base, BL)], a_v)
            pltpu.sync_copy(rip.at[pl.ds(base, BL)], b_v)
            for g in range(BL // 16):
                sl = pl.ds(g * 16, 16)
                lv = a_v[sl]
                rv = b_v[sl]
                dx = plsc.load_gather(pxv, [lv]) - plsc.load_gather(cxv, [rv])
                dy = plsc.load_gather(pyv, [lv]) - plsc.load_gather(cyv, [rv])
                dz = plsc.load_gather(pzv, [lv]) - plsc.load_gather(czv, [rv])
                ss = dx * dx + dy * dy + dz * dz + 1e-6
                yv = _rsqrt16(ss)
                f = jnp.minimum(ss * yv * jnp.float32(SCALE), HI)
                q = f.astype(jnp.int32)
                q_v[sl] = q
                t_v[sl] = f - q.astype(jnp.float32)
            osl = pl.ds(base, BL)
            pltpu.sync_copy(q_v, qc.at[osl])
            pltpu.sync_copy(t_v, tc.at[osl])
            return 0

        lax.fori_loop(0, nbc, cblock, 0)

    out_type = (
        jax.ShapeDtypeStruct((EL,), jnp.int32),
        jax.ShapeDtypeStruct((EL,), jnp.float32),
        jax.ShapeDtypeStruct((EL,), jnp.float32),
        jax.ShapeDtypeStruct((EL,), jnp.float32),
        jax.ShapeDtypeStruct((EL,), jnp.float32),
        jax.ShapeDtypeStruct((ECP,), jnp.int32),
        jax.ShapeDtypeStruct((ECP,), jnp.float32),
    )
    scratch = [
        pltpu.VMEM((N,), jnp.float32),
        pltpu.VMEM((N,), jnp.float32),
        pltpu.VMEM((N,), jnp.float32),
        pltpu.VMEM((R,), jnp.float32),
        pltpu.VMEM((R,), jnp.float32),
        pltpu.VMEM((R,), jnp.float32),
        pltpu.VMEM((QV,), jnp.float32),
        pltpu.VMEM((BL,), jnp.int32),
        pltpu.VMEM((BL,), jnp.int32),
        pltpu.VMEM((BL,), jnp.int32),
        pltpu.VMEM((BL,), jnp.float32),
        pltpu.VMEM((BL,), jnp.float32),
        pltpu.VMEM((BL,), jnp.float32),
        pltpu.VMEM((BL,), jnp.float32),
        pltpu.SemaphoreType.DMA,
    ]
    return pl.kernel(body, out_type=out_type,
                     mesh=plsc.VectorSubcoreMesh(**_MESH),
                     scratch_types=scratch, compiler_params=_NOLAYOUT)


def _lerp_mult_loop(xbuf, t0buf, t1buf, tt):
    def body(i, _):
        tv = plsc.load_gather(tt, [jnp.broadcast_to(i, (16,))])
        for j in range(D // 16):
            sl = pl.ds(j * 16, 16)
            u0 = t0buf[i, sl]
            u1 = t1buf[i, sl]
            xbuf[i, sl] = xbuf[i, sl] * (u0 + (u1 - u0) * tv)
        return 0
    lax.fori_loop(0, BL, body, 0)


def _sc_feat(cross):
    per_w = (EC // NS) if cross else (EL // (NC * NS))
    nblk = per_w // BL

    def body(feat_hbm, t0_hbm, t1_hbm, gi, si, ql, tl, z128,
             agg_out,
             agg_sh, g_v, s_v, q0_v, q1_v, tt_v, xbuf, t0buf, t1buf, sem):
        c = lax.axis_index("c")
        s = lax.axis_index("s")

        @pl.when(s < 10)
        def _init():
            isl = pl.ds(s * 1000, 1000)
            pltpu.sync_copy(z128.at[isl], agg_sh.at[isl])
        plsc.subcore_barrier()

        if cross:
            base0 = s * per_w
            qoff = c * Q
        else:
            base0 = (c * NS + s) * per_w
            qoff = 0

        def block(b, _):
            base = base0 + b * BL
            pltpu.sync_copy(gi.at[pl.ds(base, BL)], g_v)
            pltpu.sync_copy(si.at[pl.ds(base, BL)], s_v)
            pltpu.sync_copy(ql.at[pl.ds(base, BL)], q0_v)
            pltpu.sync_copy(tl.at[pl.ds(base, BL)], tt_v)
            for g in range(BL // 16):
                sl = pl.ds(g * 16, 16)
                q = q0_v[sl] + qoff
                q0_v[sl] = q
                q1_v[sl] = q + 1
            cps = [
                pltpu.async_copy(feat_hbm.at[g_v], xbuf, sem),
                pltpu.async_copy(t0_hbm.at[q0_v], t0buf, sem),
                pltpu.async_copy(t1_hbm.at[q1_v], t1buf, sem),
            ]
            for cc in cps:
                cc.wait()
            _lerp_mult_loop(xbuf, t0buf, t1buf, tt_v)
            pltpu.sync_copy(xbuf, agg_sh.at[s_v], add=True)
            return 0

        lax.fori_loop(0, nblk, block, 0)
        plsc.subcore_barrier()

        @pl.when(s < 10)
        def _dump():
            rsl = pl.ds(s * 1000, 1000)
            pltpu.sync_copy(agg_sh.at[rsl], agg_out.at[c, rsl])

    scratch = [
        pltpu.VMEM_SHARED((N, D), jnp.float32),
        pltpu.VMEM((BL,), jnp.int32),
        pltpu.VMEM((BL,), jnp.int32),
        pltpu.VMEM((BL,), jnp.int32),
        pltpu.VMEM((BL,), jnp.int32),
        pltpu.VMEM((BL,), jnp.float32),
        pltpu.VMEM((BL, D), jnp.float32),
        pltpu.VMEM((BL, D), jnp.float32),
        pltpu.VMEM((BL, D), jnp.float32),
        pltpu.SemaphoreType.DMA,
    ]
    return pl.kernel(body,
                     out_type=jax.ShapeDtypeStruct((NC, N, D), jnp.float32),
                     mesh=plsc.VectorSubcoreMesh(**_MESH),
                     scratch_types=scratch, compiler_params=_NOLAYOUT)


def _sc_vec():
    per_w = EL // (NC * NS)
    nblk = per_w // BL

    def body(vmx, vmy, vmz, si, z128,
             agg_out,
             agg_sh, s_v, mx_v, my_v, mz_v, vbuf, sem):
        c = lax.axis_index("c")
        s = lax.axis_index("s")

        @pl.when(s < 10)
        def _init():
            isl = pl.ds(s * 1000, 1000)
            pltpu.sync_copy(z128.at[isl], agg_sh.at[isl])

        def zb(i, _):
            for j in range(D // 16):
                vbuf[i, pl.ds(j * 16, 16)] = jnp.zeros((16,), jnp.float32)
            return 0
        lax.fori_loop(0, BL, zb, 0)
        plsc.subcore_barrier()

        base0 = (c * NS + s) * per_w
        lane = lax.broadcasted_iota(jnp.int32, (16,), 0)

        def block(b, _):
            base = base0 + b * BL
            pltpu.sync_copy(si.at[pl.ds(base, BL)], s_v)
            pltpu.sync_copy(vmx.at[pl.ds(base, BL)], mx_v)
            pltpu.sync_copy(vmy.at[pl.ds(base, BL)], my_v)
            pltpu.sync_copy(vmz.at[pl.ds(base, BL)], mz_v)
            zc = jnp.zeros((16,), jnp.int32)
            for g in range(BL // 16):
                sl = pl.ds(g * 16, 16)
                row = lane + g * 16
                plsc.store_scatter(vbuf, [row, zc], mx_v[sl])
                plsc.store_scatter(vbuf, [row, zc + 1], my_v[sl])
                plsc.store_scatter(vbuf, [row, zc + 2], mz_v[sl])
            pltpu.sync_copy(vbuf, agg_sh.at[s_v], add=True)
            return 0

        lax.fori_loop(0, nblk, block, 0)
        plsc.subcore_barrier()

        @pl.when(s < 10)
        def _dump():
            rsl = pl.ds(s * 1000, 1000)
            pltpu.sync_copy(agg_sh.at[rsl], agg_out.at[c, rsl])

    scratch = [
        pltpu.VMEM_SHARED((N, D), jnp.float32),
        pltpu.VMEM((BL,), jnp.int32),
        pltpu.VMEM((BL,), jnp.float32),
        pltpu.VMEM((BL,), jnp.float32),
        pltpu.VMEM((BL,), jnp.float32),
        pltpu.VMEM((BL, D), jnp.float32),
        pltpu.SemaphoreType.DMA,
    ]
    return pl.kernel(body,
                     out_type=jax.ShapeDtypeStruct((NC, N, D), jnp.float32),
                     mesh=plsc.VectorSubcoreMesh(**_MESH),
                     scratch_types=scratch, compiler_params=_NOLAYOUT)


def kernel(ligand_positions, cond_z, cond_batch, cond_mass, cond_pos_N, cond_pos_Ca, cond_pos_C, cond_residue_type, cond_batch_res, edge_index_lig, edge_index_cross_l, edge_index_cross_r, emb_atom, emb_res, W_rbf0, W_rbf1, W_upd0, W_upd1, b_upd0, b_upd1, W_vec, W_geom, b_geom, W_crbf0, W_crbf1, W_cupd0, W_cupd1, b_cupd0, b_cupd1, W_out, b_out):
    centers = jnp.linspace(0.0, CUT, NR).reshape(1, NR)
    px = ligand_positions[:, 0]
    py = ligand_positions[:, 1]
    pz = ligand_positions[:, 2]
    pcx = cond_pos_Ca[:, 0]
    pcy = cond_pos_Ca[:, 1]
    pcz = cond_pos_Ca[:, 2]
    src_l = edge_index_lig[0]
    dst_l = edge_index_lig[1]
    pad = jnp.zeros((ECP - EC,), jnp.int32)
    lip = jnp.concatenate([edge_index_cross_l, pad])
    rip = jnp.concatenate([edge_index_cross_r, pad])
    z128 = jnp.zeros((N, D), jnp.float32)

    x0 = _tc_embed(cond_z, emb_atom, N, 119, 1000)
    h = _tc_h(cond_residue_type, emb_res, cond_pos_N, cond_pos_Ca,
              cond_pos_C, W_geom, b_geom)
    t0, t1 = _tc_tab1(centers, W_rbf0, W_rbf1)
    tcc = _tc_tab2(centers, W_crbf0, W_crbf1)
    vt = _tc_vtab(centers, W_vec).reshape(QV)

    ql, tl, vmx, vmy, vmz, qc, tcf = _sc_geo()(
        px, py, pz, pcx, pcy, pcz, vt, src_l, dst_l, lip, rip)

    lig = _sc_feat(False)
    agg0 = lig(x0, t0, t0, src_l, dst_l, ql, tl, z128)
    x1 = _tc_upd(x0, agg0[0], agg0[1], W_upd0, b_upd0)
    agg1 = lig(x1, t1, t1, src_l, dst_l, ql, tl, z128)
    x2 = _tc_upd(x1, agg1[0], agg1[1], W_upd1, b_upd1)

    aggc = _sc_feat(True)(h, tcc, tcc, edge_index_cross_r,
                          edge_index_cross_l, qc, tcf, z128)

    vec = _sc_vec()(vmx, vmy, vmz, dst_l, z128)

    x4 = _tc_final(x2, aggc[0], aggc[1], vec[0], vec[1],
                   W_cupd0, b_cupd0, W_cupd1, b_cupd1)

    g = jax.ops.segment_sum(x4, cond_batch, num_segments=G)
    energy = g.mean(axis=0) @ W_out + b_out
    return energy
